# baseline (speedup 1.0000x reference)
import functools

import jax
import jax.numpy as jnp
from jax.experimental import pallas as pl
from jax.experimental.pallas import tpu as pltpu

_E = 16
_D_IN = 768
_D_HID = 1536
_D_OUT = 768
_TM = 16
_NTILES = 24
_PAD = _NTILES * _TM
_OWPAD = 32


def _bookkeep(x_ref, gw_ref, xs_ref, pt_ref, w_ref, ow_ref):
    xf = x_ref[...]
    t = xf.shape[0]
    logits = jax.lax.dot_general(
        xf, gw_ref[...], (((1,), (1,)), ((), ())),
        preferred_element_type=jnp.float32)
    m = jnp.max(logits, axis=1, keepdims=True)
    lane = jax.lax.broadcasted_iota(jnp.int32, logits.shape, 1)
    idx = jnp.min(jnp.where(logits == m, lane, _E), axis=1, keepdims=True)
    s = jnp.sum(jnp.exp(logits - m), axis=1, keepdims=True)
    w_ref[...] = 1.0 / (1.0 + 1e-8 * s)

    onehot = (lane == idx).astype(jnp.float32)
    counts = jnp.sum(onehot, axis=0, keepdims=True)
    counts_pad = jnp.floor((counts + (_TM - 1)) * (1.0 / _TM)) * _TM
    ei = jax.lax.broadcasted_iota(jnp.int32, (_E, _E), 0)
    ej = jax.lax.broadcasted_iota(jnp.int32, (_E, _E), 1)
    strict_ut = (ei < ej).astype(jnp.float32)
    offs = jax.lax.dot_general(
        counts_pad, strict_ut, (((1,), (0,)), ((), ())),
        preferred_element_type=jnp.float32)
    ti = jax.lax.broadcasted_iota(jnp.int32, (t, t), 0)
    tj = jax.lax.broadcasted_iota(jnp.int32, (t, t), 1)
    strict_lt = (tj < ti).astype(jnp.float32)
    c = jax.lax.dot_general(
        strict_lt, onehot, (((1,), (0,)), ((), ())),
        preferred_element_type=jnp.float32)
    rank = jnp.sum(c * onehot, axis=1, keepdims=True)
    dest = jnp.sum(onehot * offs, axis=1, keepdims=True) + rank

    slot = jax.lax.broadcasted_iota(jnp.int32, (t, _PAD), 1)
    pt = (dest.astype(jnp.int32) == slot).astype(jnp.float32)
    pt_ref[...] = pt
    xs_ref[...] = jax.lax.dot_general(
        pt, xf, (((0,), (0,)), ((), ())),
        preferred_element_type=jnp.float32)

    ones_col = jnp.zeros((t, 1), jnp.float32) + 1.0
    counts_col = jax.lax.dot_general(
        onehot, ones_col, (((0,), (0,)), ((), ())),
        preferred_element_type=jnp.float32)
    tiles_col = jnp.floor((counts_col + (_TM - 1)) * (1.0 / _TM))
    st_lt_e = (ej < ei).astype(jnp.float32)
    ot_col = jax.lax.dot_general(
        st_lt_e, tiles_col, (((1,), (0,)), ((), ())),
        preferred_element_type=jnp.float32)
    ends_col = ot_col + tiles_col
    k_row = jax.lax.broadcasted_iota(jnp.int32, (1, _OWPAD), 1)
    ow = jnp.sum((ends_col <= k_row.astype(jnp.float32)).astype(jnp.float32),
                 axis=0, keepdims=True)
    ow_ref[...] = jnp.minimum(ow, jnp.float32(_E - 1)).astype(jnp.int32)


def _expert_tile(ow_ref, xs_ref, w1_ref, b1_ref, w2_ref, b2_ref,
                 pt_ref, wcol_ref, x_ref, out_ref, ys_ref):
    k = pl.program_id(0)
    rows = xs_ref[0]
    h = jax.lax.dot_general(
        rows, w1_ref[0], (((1,), (1,)), ((), ())),
        preferred_element_type=jnp.float32)
    h = jnp.maximum(h + b1_ref[0], 0.0)
    y = jax.lax.dot_general(
        h, w2_ref[0], (((1,), (1,)), ((), ())),
        preferred_element_type=jnp.float32)
    ys_ref[k] = y + b2_ref[0]

    @pl.when(k == _NTILES - 1)
    def _finish():
        ys = ys_ref[...].reshape(_PAD, _D_OUT)
        unsorted = jax.lax.dot_general(
            pt_ref[...], ys, (((1,), (0,)), ((), ())),
            preferred_element_type=jnp.float32)
        out_ref[...] = wcol_ref[...] * unsorted + x_ref[...]


@functools.partial(jax.jit, static_argnames=("interpret",))
def kernel(x, gate_w, W1, b1, W2, b2, interpret=False):
    orig_shape = x.shape
    xf = x.reshape(-1, orig_shape[-1])
    t = xf.shape[0]

    xs, pt, wcol, ow = pl.pallas_call(
        _bookkeep,
        grid=(1,),
        in_specs=[
            pl.BlockSpec((t, _D_IN), lambda i: (0, 0)),
            pl.BlockSpec((_E, _D_IN), lambda i: (0, 0)),
        ],
        out_specs=[
            pl.BlockSpec((_PAD, _D_IN), lambda i: (0, 0)),
            pl.BlockSpec((t, _PAD), lambda i: (0, 0)),
            pl.BlockSpec((t, 1), lambda i: (0, 0)),
            pl.BlockSpec((1, _OWPAD), lambda i: (0, 0)),
        ],
        out_shape=[
            jax.ShapeDtypeStruct((_PAD, _D_IN), jnp.float32),
            jax.ShapeDtypeStruct((t, _PAD), jnp.float32),
            jax.ShapeDtypeStruct((t, 1), jnp.float32),
            jax.ShapeDtypeStruct((1, _OWPAD), jnp.int32),
        ],
        interpret=interpret,
    )(xf, gate_w)

    xs3 = xs.reshape(_NTILES, _TM, _D_IN)
    ow1 = ow.reshape(_OWPAD)

    out = pl.pallas_call(
        _expert_tile,
        grid_spec=pltpu.PrefetchScalarGridSpec(
            num_scalar_prefetch=1,
            grid=(_NTILES,),
            in_specs=[
                pl.BlockSpec((1, _TM, _D_IN), lambda k, ow: (k, 0, 0)),
                pl.BlockSpec((1, _D_HID, _D_IN), lambda k, ow: (ow[k], 0, 0)),
                pl.BlockSpec((1, 1, _D_HID), lambda k, ow: (ow[k], 0, 0)),
                pl.BlockSpec((1, _D_OUT, _D_HID), lambda k, ow: (ow[k], 0, 0)),
                pl.BlockSpec((1, 1, _D_OUT), lambda k, ow: (ow[k], 0, 0)),
                pl.BlockSpec((t, _PAD), lambda k, ow: (0, 0)),
                pl.BlockSpec((t, 1), lambda k, ow: (0, 0)),
                pl.BlockSpec((t, _D_IN), lambda k, ow: (0, 0)),
            ],
            out_specs=pl.BlockSpec((t, _D_OUT), lambda k, ow: (0, 0)),
            scratch_shapes=[pltpu.VMEM((_NTILES, _TM, _D_OUT), jnp.float32)],
        ),
        out_shape=jax.ShapeDtypeStruct((t, _D_OUT), jnp.float32),
        interpret=interpret,
    )(ow1, xs3, W1, b1[:, None, :], W2, b2[:, None, :], pt, wcol, xf)

    return out.reshape(orig_shape[:-1] + (_D_OUT,))


# speedup vs baseline: 1.2070x; 1.2070x over previous
import jax
import jax.numpy as jnp
from jax.experimental import pallas as pl
from jax.experimental.pallas import tpu as pltpu

_E = 16
_D_IN = 768
_D_HID = 1536
_D_OUT = 768


def _moe_step(x_ref, gw_ref, w1_ref, b1_ref, w2_ref, b2_ref, out_ref,
              widx_ref, wcol_ref):
    e = pl.program_id(0)
    xf = x_ref[...]

    @pl.when(e == 0)
    def _router():
        logits = jax.lax.dot_general(
            xf, gw_ref[...], (((1,), (1,)), ((), ())),
            preferred_element_type=jnp.float32)
        m = jnp.max(logits, axis=1, keepdims=True)
        lane = jax.lax.broadcasted_iota(jnp.int32, logits.shape, 1)
        idx = jnp.min(jnp.where(logits == m, lane, _E),
                      axis=1, keepdims=True).astype(jnp.float32)
        s = jnp.sum(jnp.exp(logits - m), axis=1, keepdims=True)
        widx_ref[...] = idx
        wcol_ref[...] = 1.0 / (1.0 + 1e-8 * s)

    h = jax.lax.dot_general(
        xf, w1_ref[0], (((1,), (1,)), ((), ())),
        preferred_element_type=jnp.float32)
    h = jnp.maximum(h + b1_ref[0], 0.0)
    y = jax.lax.dot_general(
        h, w2_ref[0], (((1,), (1,)), ((), ())),
        preferred_element_type=jnp.float32)
    y = y + b2_ref[0]

    gate = jnp.where(widx_ref[...] == jnp.float32(1) * e, wcol_ref[...], 0.0)
    contrib = gate * y

    @pl.when(e == 0)
    def _init():
        out_ref[...] = xf + contrib

    @pl.when(e != 0)
    def _acc():
        out_ref[...] += contrib


@jax.jit
def kernel(x, gate_w, W1, b1, W2, b2):
    orig_shape = x.shape
    xf = x.reshape(-1, orig_shape[-1])
    t = xf.shape[0]

    out = pl.pallas_call(
        _moe_step,
        grid=(_E,),
        in_specs=[
            pl.BlockSpec((t, _D_IN), lambda e: (0, 0)),
            pl.BlockSpec((_E, _D_IN), lambda e: (0, 0)),
            pl.BlockSpec((1, _D_HID, _D_IN), lambda e: (e, 0, 0)),
            pl.BlockSpec((1, 1, _D_HID), lambda e: (e, 0, 0)),
            pl.BlockSpec((1, _D_OUT, _D_HID), lambda e: (e, 0, 0)),
            pl.BlockSpec((1, 1, _D_OUT), lambda e: (e, 0, 0)),
        ],
        out_specs=pl.BlockSpec((t, _D_OUT), lambda e: (0, 0)),
        out_shape=jax.ShapeDtypeStruct((t, _D_OUT), jnp.float32),
        scratch_shapes=[
            pltpu.VMEM((t, 1), jnp.float32),
            pltpu.VMEM((t, 1), jnp.float32),
        ],
    )(xf, gate_w, W1, b1[:, None, :], W2, b2[:, None, :])

    return out.reshape(orig_shape[:-1] + (_D_OUT,))
